# baseline (speedup 1.0000x reference)
import functools

import jax
import jax.numpy as jnp
from jax.experimental import pallas as pl
from jax.experimental.pallas import tpu as pltpu

B = 16
N = 2048
NP1, K1 = 512, 16
NP2, K2 = 256, 16


def _fps_kernel(xyz_ref, new_xyz_ref, *, npoint):
    x = xyz_ref[:, 0, :]
    y = xyz_ref[:, 1, :]
    z = xyz_ref[:, 2, :]
    Bn, Nn = x.shape
    iota = jax.lax.broadcasted_iota(jnp.int32, (Bn, Nn), 1)
    CHUNK = 128
    iota_c = jax.lax.broadcasted_iota(jnp.int32, (Bn, CHUNK), 1)

    def body(j, state):
        dists, far, bx, by, bz = state
        onehot = (iota == far)
        cx = jnp.sum(jnp.where(onehot, x, 0.0), axis=1, keepdims=True)
        cy = jnp.sum(jnp.where(onehot, y, 0.0), axis=1, keepdims=True)
        cz = jnp.sum(jnp.where(onehot, z, 0.0), axis=1, keepdims=True)
        sel = iota_c == j
        bx = jnp.where(sel, cx, bx)
        by = jnp.where(sel, cy, by)
        bz = jnp.where(sel, cz, bz)
        dx = x - cx
        dy = y - cy
        dz = z - cz
        d = dx * dx + dy * dy + dz * dz
        dists = jnp.minimum(dists, d)
        m = jnp.max(dists, axis=1, keepdims=True)
        far = jnp.min(jnp.where(dists == m, iota, Nn), axis=1, keepdims=True)
        return dists, far.astype(jnp.int32), bx, by, bz

    dists = jnp.full((Bn, Nn), 1e10, dtype=jnp.float32)
    far = jnp.zeros((Bn, 1), dtype=jnp.int32)
    zblk = jnp.zeros((Bn, CHUNK), dtype=jnp.float32)
    for c in range(npoint // CHUNK):
        dists, far, bx, by, bz = jax.lax.fori_loop(
            0, CHUNK, body, (dists, far, zblk, zblk, zblk))
        new_xyz_ref[:, 0, c * CHUNK:(c + 1) * CHUNK] = bx
        new_xyz_ref[:, 1, c * CHUNK:(c + 1) * CHUNK] = by
        new_xyz_ref[:, 2, c * CHUNK:(c + 1) * CHUNK] = bz


def _fps_pallas(xyz, npoint):
    Bn, _, Nn = xyz.shape
    return pl.pallas_call(
        functools.partial(_fps_kernel, npoint=npoint),
        out_shape=jax.ShapeDtypeStruct((Bn, 3, npoint), jnp.float32),
        in_specs=[pl.BlockSpec((Bn, 3, Nn), lambda: (0, 0, 0))],
        out_specs=pl.BlockSpec((Bn, 3, npoint), lambda: (0, 0, 0)),
    )(xyz)


def _mlp_pool_kernel(g_ref, wa_ref, ba_ref, wb_ref, bb_ref, out_ref, *, k):
    wa = wa_ref[...]
    ba = ba_ref[...]
    wb = wb_ref[...]
    bb = bb_ref[...]
    acc = None
    for j in range(k):
        xj = g_ref[0, j]
        h = jnp.dot(wa, xj, preferred_element_type=jnp.float32) + ba
        h = jnp.maximum(h, 0.0)
        h2 = jnp.dot(wb, h, preferred_element_type=jnp.float32) + bb
        acc = h2 if acc is None else jnp.maximum(acc, h2)
    out_ref[0] = acc


def _mlp_pool_pallas(grouped, wa, ba, wb, bb):
    Bn, K, C_in, M = grouped.shape
    C_hid = wa.shape[0]
    C_out = wb.shape[0]
    ba2 = ba.reshape(C_hid, 1)
    bb2 = bb.reshape(C_out, 1)
    return pl.pallas_call(
        functools.partial(_mlp_pool_kernel, k=K),
        out_shape=jax.ShapeDtypeStruct((Bn, C_out, M), jnp.float32),
        grid=(Bn,),
        in_specs=[
            pl.BlockSpec((1, K, C_in, M), lambda b: (b, 0, 0, 0)),
            pl.BlockSpec((C_hid, C_in), lambda b: (0, 0)),
            pl.BlockSpec((C_hid, 1), lambda b: (0, 0)),
            pl.BlockSpec((C_out, C_hid), lambda b: (0, 0)),
            pl.BlockSpec((C_out, 1), lambda b: (0, 0)),
        ],
        out_specs=pl.BlockSpec((1, C_out, M), lambda b: (b, 0, 0)),
    )(grouped, wa, ba2, wb, bb2)


def _grouping(points, idx):
    Bn, C, _ = points.shape
    _, M, K = idx.shape
    flat = jnp.broadcast_to(idx.reshape(Bn, 1, M * K), (Bn, C, M * K))
    g = jnp.take_along_axis(points, flat, axis=2)
    return g.reshape(Bn, C, M, K)


def _sa_module(xyz, points, npoint, k, wa, ba, wb, bb):
    new_xyz = _fps_pallas(xyz, npoint)
    xyz_t = jnp.transpose(xyz, (0, 2, 1))
    new_xyz_t = jnp.transpose(new_xyz, (0, 2, 1))
    dist = jnp.sum((new_xyz_t[:, :, None, :] - xyz_t[:, None, :, :]) ** 2, axis=-1)
    _, idx = jax.lax.top_k(-dist, k)
    idx = idx.astype(jnp.int32)
    grouped_xyz = _grouping(xyz, idx) - new_xyz[:, :, :, None]
    grouped_points = _grouping(points, idx)
    new_points = jnp.concatenate([grouped_xyz, grouped_points], axis=1)
    g = jnp.transpose(new_points, (0, 3, 1, 2))
    out = _mlp_pool_pallas(g, wa, ba, wb, bb)
    return new_xyz, out


def kernel(point_cloud, W1, b1, W2, b2, W3, b3, W4, b4):
    l1_xyz, l1_points = _sa_module(point_cloud, point_cloud, NP1, K1, W1, b1, W2, b2)
    _, l2_points = _sa_module(l1_xyz, l1_points, NP2, K2, W3, b3, W4, b4)
    return l2_points


# speedup vs baseline: 47.3178x; 47.3178x over previous
import functools

import jax
import jax.numpy as jnp
from jax.experimental import pallas as pl
from jax.experimental.pallas import tpu as pltpu

B = 16
N = 2048
NP1, K1 = 512, 16
NP2, K2 = 256, 16


def _fps_kernel(xyz_ref, new_xyz_ref, *, npoint):
    x = xyz_ref[:, 0, :]
    y = xyz_ref[:, 1, :]
    z = xyz_ref[:, 2, :]
    Bn, Nn = x.shape
    iota = jax.lax.broadcasted_iota(jnp.int32, (Bn, Nn), 1)
    CHUNK = 128
    iota_c = jax.lax.broadcasted_iota(jnp.int32, (Bn, CHUNK), 1)

    def body(j, state):
        dists, far, bx, by, bz = state
        onehot = (iota == far)
        cx = jnp.sum(jnp.where(onehot, x, 0.0), axis=1, keepdims=True)
        cy = jnp.sum(jnp.where(onehot, y, 0.0), axis=1, keepdims=True)
        cz = jnp.sum(jnp.where(onehot, z, 0.0), axis=1, keepdims=True)
        sel = iota_c == j
        bx = jnp.where(sel, cx, bx)
        by = jnp.where(sel, cy, by)
        bz = jnp.where(sel, cz, bz)
        dx = x - cx
        dy = y - cy
        dz = z - cz
        d = dx * dx + dy * dy + dz * dz
        dists = jnp.minimum(dists, d)
        m = jnp.max(dists, axis=1, keepdims=True)
        far = jnp.min(jnp.where(dists == m, iota, Nn), axis=1, keepdims=True)
        return dists, far.astype(jnp.int32), bx, by, bz

    dists = jnp.full((Bn, Nn), 1e10, dtype=jnp.float32)
    far = jnp.zeros((Bn, 1), dtype=jnp.int32)
    zblk = jnp.zeros((Bn, CHUNK), dtype=jnp.float32)
    for c in range(npoint // CHUNK):
        dists, far, bx, by, bz = jax.lax.fori_loop(
            0, CHUNK, body, (dists, far, zblk, zblk, zblk))
        new_xyz_ref[:, 0, c * CHUNK:(c + 1) * CHUNK] = bx
        new_xyz_ref[:, 1, c * CHUNK:(c + 1) * CHUNK] = by
        new_xyz_ref[:, 2, c * CHUNK:(c + 1) * CHUNK] = bz


def _fps_pallas(xyz, npoint):
    Bn, _, Nn = xyz.shape
    return pl.pallas_call(
        functools.partial(_fps_kernel, npoint=npoint),
        out_shape=jax.ShapeDtypeStruct((Bn, 3, npoint), jnp.float32),
        in_specs=[pl.BlockSpec((Bn, 3, Nn), lambda: (0, 0, 0))],
        out_specs=pl.BlockSpec((Bn, 3, npoint), lambda: (0, 0, 0)),
    )(xyz)


def _mlp_pool_kernel(g_ref, wa_ref, ba_ref, wb_ref, bb_ref, out_ref, *, k):
    wa = wa_ref[...]
    ba = ba_ref[...]
    wb = wb_ref[...]
    bb = bb_ref[...]
    acc = None
    for j in range(k):
        xj = g_ref[j, 0]
        h = jnp.dot(wa, xj, preferred_element_type=jnp.float32) + ba
        h = jnp.maximum(h, 0.0)
        h2 = jnp.dot(wb, h, preferred_element_type=jnp.float32) + bb
        acc = h2 if acc is None else jnp.maximum(acc, h2)
    out_ref[0] = acc


def _mlp_pool_pallas(grouped, wa, ba, wb, bb):
    K, Bn, C_in, M = grouped.shape
    C_hid = wa.shape[0]
    C_out = wb.shape[0]
    ba2 = ba.reshape(C_hid, 1)
    bb2 = bb.reshape(C_out, 1)
    return pl.pallas_call(
        functools.partial(_mlp_pool_kernel, k=K),
        out_shape=jax.ShapeDtypeStruct((Bn, C_out, M), jnp.float32),
        grid=(Bn,),
        in_specs=[
            pl.BlockSpec((K, 1, C_in, M), lambda b: (0, b, 0, 0)),
            pl.BlockSpec((C_hid, C_in), lambda b: (0, 0)),
            pl.BlockSpec((C_hid, 1), lambda b: (0, 0)),
            pl.BlockSpec((C_out, C_hid), lambda b: (0, 0)),
            pl.BlockSpec((C_out, 1), lambda b: (0, 0)),
        ],
        out_specs=pl.BlockSpec((1, C_out, M), lambda b: (b, 0, 0)),
    )(grouped, wa, ba2, wb, bb2)


def _sa_module(xyz, points, npoint, k, wa, ba, wb, bb):
    Bn, C, Nn = points.shape
    new_xyz = _fps_pallas(xyz, npoint)
    xyz_t = jnp.transpose(xyz, (0, 2, 1))
    new_xyz_t = jnp.transpose(new_xyz, (0, 2, 1))
    dist = jnp.sum((new_xyz_t[:, :, None, :] - xyz_t[:, None, :, :]) ** 2, axis=-1)
    _, idx = jax.lax.top_k(-dist, k)
    idx = idx.astype(jnp.int32)
    slabs = []
    for j in range(k):
        ij = idx[:, :, j]
        gx = jnp.take_along_axis(
            xyz, jnp.broadcast_to(ij[:, None, :], (Bn, 3, npoint)), axis=2)
        gx = gx - new_xyz
        gp = jnp.take_along_axis(
            points, jnp.broadcast_to(ij[:, None, :], (Bn, C, npoint)), axis=2)
        slabs.append(jnp.concatenate([gx, gp], axis=1))
    g = jnp.stack(slabs, axis=0)
    out = _mlp_pool_pallas(g, wa, ba, wb, bb)
    return new_xyz, out


def kernel(point_cloud, W1, b1, W2, b2, W3, b3, W4, b4):
    l1_xyz, l1_points = _sa_module(point_cloud, point_cloud, NP1, K1, W1, b1, W2, b2)
    _, l2_points = _sa_module(l1_xyz, l1_points, NP2, K2, W3, b3, W4, b4)
    return l2_points


# speedup vs baseline: 47.6778x; 1.0076x over previous
import functools

import jax
import jax.numpy as jnp
from jax.experimental import pallas as pl
from jax.experimental.pallas import tpu as pltpu

B = 16
N = 2048
NP1, K1 = 512, 16
NP2, K2 = 256, 16


def _fps_kernel(xyz_ref, new_xyz_ref, *, npoint):
    x = xyz_ref[:, 0, :]
    y = xyz_ref[:, 1, :]
    z = xyz_ref[:, 2, :]
    Bn, Nn = x.shape
    iota = jax.lax.broadcasted_iota(jnp.int32, (Bn, Nn), 1)
    CHUNK = 128
    iota_c = jax.lax.broadcasted_iota(jnp.int32, (Bn, CHUNK), 1)

    def body(j, state):
        dists, far, bx, by, bz = state
        onehot = (iota == far)
        cx = jnp.sum(jnp.where(onehot, x, 0.0), axis=1, keepdims=True)
        cy = jnp.sum(jnp.where(onehot, y, 0.0), axis=1, keepdims=True)
        cz = jnp.sum(jnp.where(onehot, z, 0.0), axis=1, keepdims=True)
        sel = iota_c == j
        bx = jnp.where(sel, cx, bx)
        by = jnp.where(sel, cy, by)
        bz = jnp.where(sel, cz, bz)
        dx = x - cx
        dy = y - cy
        dz = z - cz
        d = dx * dx + dy * dy + dz * dz
        dists = jnp.minimum(dists, d)
        m = jnp.max(dists, axis=1, keepdims=True)
        far = jnp.min(jnp.where(dists == m, iota, Nn), axis=1, keepdims=True)
        return dists, far.astype(jnp.int32), bx, by, bz

    dists = jnp.full((Bn, Nn), 1e10, dtype=jnp.float32)
    far = jnp.zeros((Bn, 1), dtype=jnp.int32)
    zblk = jnp.zeros((Bn, CHUNK), dtype=jnp.float32)
    for c in range(npoint // CHUNK):
        dists, far, bx, by, bz = jax.lax.fori_loop(
            0, CHUNK, body, (dists, far, zblk, zblk, zblk))
        new_xyz_ref[:, 0, c * CHUNK:(c + 1) * CHUNK] = bx
        new_xyz_ref[:, 1, c * CHUNK:(c + 1) * CHUNK] = by
        new_xyz_ref[:, 2, c * CHUNK:(c + 1) * CHUNK] = bz


def _fps_pallas(xyz, npoint):
    Bn, _, Nn = xyz.shape
    return pl.pallas_call(
        functools.partial(_fps_kernel, npoint=npoint),
        out_shape=jax.ShapeDtypeStruct((Bn, 3, npoint), jnp.float32),
        in_specs=[pl.BlockSpec((Bn, 3, Nn), lambda: (0, 0, 0))],
        out_specs=pl.BlockSpec((Bn, 3, npoint), lambda: (0, 0, 0)),
    )(xyz)


def _mlp_pool_kernel(g_ref, wa_ref, ba_ref, wb_ref, bb_ref, out_ref, *, k):
    wa = wa_ref[...]
    ba = ba_ref[...]
    wb = wb_ref[...]
    bb = bb_ref[...]
    acc = None
    for j in range(k):
        xj = g_ref[j, 0]
        h = jnp.dot(wa, xj, preferred_element_type=jnp.float32) + ba
        h = jnp.maximum(h, 0.0)
        h2 = jnp.dot(wb, h, preferred_element_type=jnp.float32) + bb
        acc = h2 if acc is None else jnp.maximum(acc, h2)
    out_ref[0] = acc


def _mlp_pool_pallas(grouped, wa, ba, wb, bb):
    K, Bn, C_in, M = grouped.shape
    C_hid = wa.shape[0]
    C_out = wb.shape[0]
    ba2 = ba.reshape(C_hid, 1)
    bb2 = bb.reshape(C_out, 1)
    return pl.pallas_call(
        functools.partial(_mlp_pool_kernel, k=K),
        out_shape=jax.ShapeDtypeStruct((Bn, C_out, M), jnp.float32),
        grid=(Bn,),
        in_specs=[
            pl.BlockSpec((K, 1, C_in, M), lambda b: (0, b, 0, 0)),
            pl.BlockSpec((C_hid, C_in), lambda b: (0, 0)),
            pl.BlockSpec((C_hid, 1), lambda b: (0, 0)),
            pl.BlockSpec((C_out, C_hid), lambda b: (0, 0)),
            pl.BlockSpec((C_out, 1), lambda b: (0, 0)),
        ],
        out_specs=pl.BlockSpec((1, C_out, M), lambda b: (b, 0, 0)),
    )(grouped, wa, ba2, wb, bb2)


def _sa_module(xyz, points, npoint, k, wa, ba, wb, bb):
    Bn, C, Nn = points.shape
    new_xyz = _fps_pallas(xyz, npoint)
    xyz_t = jnp.transpose(xyz, (0, 2, 1))
    new_xyz_t = jnp.transpose(new_xyz, (0, 2, 1))
    dist = jnp.sum((new_xyz_t[:, :, None, :] - xyz_t[:, None, :, :]) ** 2, axis=-1)
    _, idx = jax.lax.top_k(-dist, k)
    idx = idx.astype(jnp.int32)
    P = jnp.concatenate([xyz, points], axis=1)
    slabs = []
    for j in range(k):
        ij = idx[:, :, j]
        gp = jnp.take_along_axis(
            P, jnp.broadcast_to(ij[:, None, :], (Bn, 3 + C, npoint)), axis=2)
        slabs.append(jnp.concatenate([gp[:, :3] - new_xyz, gp[:, 3:]], axis=1))
    g = jnp.stack(slabs, axis=0)
    out = _mlp_pool_pallas(g, wa, ba, wb, bb)
    return new_xyz, out


def kernel(point_cloud, W1, b1, W2, b2, W3, b3, W4, b4):
    l1_xyz, l1_points = _sa_module(point_cloud, point_cloud, NP1, K1, W1, b1, W2, b2)
    _, l2_points = _sa_module(l1_xyz, l1_points, NP2, K2, W3, b3, W4, b4)
    return l2_points


# speedup vs baseline: 129.2605x; 2.7111x over previous
import functools

import jax
import jax.numpy as jnp
from jax.experimental import pallas as pl
from jax.experimental.pallas import tpu as pltpu

B = 16
N = 2048
NP1, K1 = 512, 16
NP2, K2 = 256, 16


def _fps_kernel(xyz_ref, new_xyz_ref, *, npoint):
    x = xyz_ref[:, 0, :]
    y = xyz_ref[:, 1, :]
    z = xyz_ref[:, 2, :]
    Bn, Nn = x.shape
    iota = jax.lax.broadcasted_iota(jnp.int32, (Bn, Nn), 1)
    CHUNK = 128
    iota_c = jax.lax.broadcasted_iota(jnp.int32, (Bn, CHUNK), 1)

    def body(j, state):
        dists, far, bx, by, bz = state
        onehot = (iota == far)
        cx = jnp.sum(jnp.where(onehot, x, 0.0), axis=1, keepdims=True)
        cy = jnp.sum(jnp.where(onehot, y, 0.0), axis=1, keepdims=True)
        cz = jnp.sum(jnp.where(onehot, z, 0.0), axis=1, keepdims=True)
        sel = iota_c == j
        bx = jnp.where(sel, cx, bx)
        by = jnp.where(sel, cy, by)
        bz = jnp.where(sel, cz, bz)
        dx = x - cx
        dy = y - cy
        dz = z - cz
        d = dx * dx + dy * dy + dz * dz
        dists = jnp.minimum(dists, d)
        m = jnp.max(dists, axis=1, keepdims=True)
        far = jnp.min(jnp.where(dists == m, iota, Nn), axis=1, keepdims=True)
        return dists, far.astype(jnp.int32), bx, by, bz

    dists = jnp.full((Bn, Nn), 1e10, dtype=jnp.float32)
    far = jnp.zeros((Bn, 1), dtype=jnp.int32)
    zblk = jnp.zeros((Bn, CHUNK), dtype=jnp.float32)
    for c in range(npoint // CHUNK):
        dists, far, bx, by, bz = jax.lax.fori_loop(
            0, CHUNK, body, (dists, far, zblk, zblk, zblk))
        new_xyz_ref[:, 0, c * CHUNK:(c + 1) * CHUNK] = bx
        new_xyz_ref[:, 1, c * CHUNK:(c + 1) * CHUNK] = by
        new_xyz_ref[:, 2, c * CHUNK:(c + 1) * CHUNK] = bz


def _fps_pallas(xyz, npoint):
    Bn, _, Nn = xyz.shape
    return pl.pallas_call(
        functools.partial(_fps_kernel, npoint=npoint),
        out_shape=jax.ShapeDtypeStruct((Bn, 3, npoint), jnp.float32),
        in_specs=[pl.BlockSpec((Bn, 3, Nn), lambda: (0, 0, 0))],
        out_specs=pl.BlockSpec((Bn, 3, npoint), lambda: (0, 0, 0)),
    )(xyz)


def _knn_kernel(nxt_ref, xyz_ref, idx_ref, d_ref, *, k):
    cx = nxt_ref[0, :, 0:1]
    cy = nxt_ref[0, :, 1:2]
    cz = nxt_ref[0, :, 2:3]
    px = xyz_ref[0, 0:1, :]
    py = xyz_ref[0, 1:2, :]
    pz = xyz_ref[0, 2:3, :]
    dx = cx - px
    dy = cy - py
    dz = cz - pz
    d_ref[...] = dx * dx + dy * dy + dz * dz
    M, N = d_ref.shape
    iota = jax.lax.broadcasted_iota(jnp.int32, (M, N), 1)
    iota_k = jax.lax.broadcasted_iota(jnp.int32, (M, k), 1)
    acc = jnp.zeros((M, k), jnp.int32)
    for j in range(k):
        d = d_ref[...]
        m = jnp.min(d, axis=1, keepdims=True)
        cand = jnp.where(d == m, iota, N)
        idxm = jnp.min(cand, axis=1, keepdims=True)
        acc = jnp.where(iota_k == j, idxm, acc)
        d_ref[...] = jnp.where(iota == idxm, jnp.inf, d)
    idx_ref[0] = acc


def _knn_pallas(new_xyz, xyz, k):
    Bn, _, M = new_xyz.shape
    N = xyz.shape[2]
    nxt = jnp.transpose(new_xyz, (0, 2, 1))
    return pl.pallas_call(
        functools.partial(_knn_kernel, k=k),
        out_shape=jax.ShapeDtypeStruct((Bn, M, k), jnp.int32),
        grid=(Bn,),
        in_specs=[
            pl.BlockSpec((1, M, 3), lambda b: (b, 0, 0)),
            pl.BlockSpec((1, 3, N), lambda b: (b, 0, 0)),
        ],
        out_specs=pl.BlockSpec((1, M, k), lambda b: (b, 0, 0)),
        scratch_shapes=[pltpu.VMEM((M, N), jnp.float32)],
    )(nxt, xyz)


def _mlp_pool_kernel(g_ref, wa_ref, ba_ref, wb_ref, bb_ref, out_ref, *, k):
    wa = wa_ref[...]
    ba = ba_ref[...]
    wb = wb_ref[...]
    bb = bb_ref[...]
    acc = None
    for j in range(k):
        xj = g_ref[j, 0]
        h = jnp.dot(wa, xj, preferred_element_type=jnp.float32) + ba
        h = jnp.maximum(h, 0.0)
        h2 = jnp.dot(wb, h, preferred_element_type=jnp.float32) + bb
        acc = h2 if acc is None else jnp.maximum(acc, h2)
    out_ref[0] = acc


def _mlp_pool_pallas(grouped, wa, ba, wb, bb):
    K, Bn, C_in, M = grouped.shape
    C_hid = wa.shape[0]
    C_out = wb.shape[0]
    ba2 = ba.reshape(C_hid, 1)
    bb2 = bb.reshape(C_out, 1)
    return pl.pallas_call(
        functools.partial(_mlp_pool_kernel, k=K),
        out_shape=jax.ShapeDtypeStruct((Bn, C_out, M), jnp.float32),
        grid=(Bn,),
        in_specs=[
            pl.BlockSpec((K, 1, C_in, M), lambda b: (0, b, 0, 0)),
            pl.BlockSpec((C_hid, C_in), lambda b: (0, 0)),
            pl.BlockSpec((C_hid, 1), lambda b: (0, 0)),
            pl.BlockSpec((C_out, C_hid), lambda b: (0, 0)),
            pl.BlockSpec((C_out, 1), lambda b: (0, 0)),
        ],
        out_specs=pl.BlockSpec((1, C_out, M), lambda b: (b, 0, 0)),
    )(grouped, wa, ba2, wb, bb2)


def _sa_module(xyz, points, npoint, k, wa, ba, wb, bb):
    Bn, C, Nn = points.shape
    new_xyz = _fps_pallas(xyz, npoint)
    idx = _knn_pallas(new_xyz, xyz, k)
    P = jnp.concatenate([xyz, points], axis=1)
    slabs = []
    for j in range(k):
        ij = idx[:, :, j]
        gp = jnp.take_along_axis(
            P, jnp.broadcast_to(ij[:, None, :], (Bn, 3 + C, npoint)), axis=2)
        slabs.append(jnp.concatenate([gp[:, :3] - new_xyz, gp[:, 3:]], axis=1))
    g = jnp.stack(slabs, axis=0)
    out = _mlp_pool_pallas(g, wa, ba, wb, bb)
    return new_xyz, out


def kernel(point_cloud, W1, b1, W2, b2, W3, b3, W4, b4):
    l1_xyz, l1_points = _sa_module(point_cloud, point_cloud, NP1, K1, W1, b1, W2, b2)
    _, l2_points = _sa_module(l1_xyz, l1_points, NP2, K2, W3, b3, W4, b4)
    return l2_points


# speedup vs baseline: 209.9580x; 1.6243x over previous
import functools

import jax
import jax.numpy as jnp
from jax.experimental import pallas as pl
from jax.experimental.pallas import tpu as pltpu

B = 16
N = 2048
NP1, K1 = 512, 16
NP2, K2 = 256, 16


def _fps_kernel(xyz_ref, new_xyz_ref, *, npoint):
    x = xyz_ref[:, 0, :]
    y = xyz_ref[:, 1, :]
    z = xyz_ref[:, 2, :]
    Bn, Nn = x.shape
    iota = jax.lax.broadcasted_iota(jnp.int32, (Bn, Nn), 1)
    CHUNK = 128
    iota_c = jax.lax.broadcasted_iota(jnp.int32, (Bn, CHUNK), 1)

    def body(j, state):
        dists, far, bx, by, bz = state
        onehot = (iota == far)
        cx = jnp.sum(jnp.where(onehot, x, 0.0), axis=1, keepdims=True)
        cy = jnp.sum(jnp.where(onehot, y, 0.0), axis=1, keepdims=True)
        cz = jnp.sum(jnp.where(onehot, z, 0.0), axis=1, keepdims=True)
        sel = iota_c == j
        bx = jnp.where(sel, cx, bx)
        by = jnp.where(sel, cy, by)
        bz = jnp.where(sel, cz, bz)
        dx = x - cx
        dy = y - cy
        dz = z - cz
        d = dx * dx + dy * dy + dz * dz
        dists = jnp.minimum(dists, d)
        m = jnp.max(dists, axis=1, keepdims=True)
        far = jnp.min(jnp.where(dists == m, iota, Nn), axis=1, keepdims=True)
        return dists, far.astype(jnp.int32), bx, by, bz

    dists = jnp.full((Bn, Nn), 1e10, dtype=jnp.float32)
    far = jnp.zeros((Bn, 1), dtype=jnp.int32)
    zblk = jnp.zeros((Bn, CHUNK), dtype=jnp.float32)
    for c in range(npoint // CHUNK):
        dists, far, bx, by, bz = jax.lax.fori_loop(
            0, CHUNK, body, (dists, far, zblk, zblk, zblk))
        new_xyz_ref[:, 0, c * CHUNK:(c + 1) * CHUNK] = bx
        new_xyz_ref[:, 1, c * CHUNK:(c + 1) * CHUNK] = by
        new_xyz_ref[:, 2, c * CHUNK:(c + 1) * CHUNK] = bz


def _fps_pallas(xyz, npoint):
    Bn, _, Nn = xyz.shape
    return pl.pallas_call(
        functools.partial(_fps_kernel, npoint=npoint),
        out_shape=jax.ShapeDtypeStruct((Bn, 3, npoint), jnp.float32),
        in_specs=[pl.BlockSpec((Bn, 3, Nn), lambda: (0, 0, 0))],
        out_specs=pl.BlockSpec((Bn, 3, npoint), lambda: (0, 0, 0)),
    )(xyz)


def _sa_kernel(nxt_ref, xyz_ref, xyzt_ref, ptst_ref,
               wat_x_ref, wat_f_ref, ba_ref, wbt_ref, bb_ref,
               out_ref, d_ref, *, k, feat_is_xyz):
    cx = nxt_ref[0, :, 0:1]
    cy = nxt_ref[0, :, 1:2]
    cz = nxt_ref[0, :, 2:3]
    px = xyz_ref[0, 0:1, :]
    py = xyz_ref[0, 1:2, :]
    pz = xyz_ref[0, 2:3, :]
    dx = cx - px
    dy = cy - py
    dz = cz - pz
    d_ref[...] = dx * dx + dy * dy + dz * dz
    M, Nn = d_ref.shape
    iota = jax.lax.broadcasted_iota(jnp.int32, (M, Nn), 1)
    xyzt = xyzt_ref[0]
    c_m = nxt_ref[0]
    wat_x = wat_x_ref[...]
    wat_f = wat_f_ref[...]
    ba = ba_ref[...]
    wbt = wbt_ref[...]
    bb = bb_ref[...]
    acc = None
    for j in range(k):
        d = d_ref[...]
        m = jnp.min(d, axis=1, keepdims=True)
        idxm = jnp.min(jnp.where(d == m, iota, Nn), axis=1, keepdims=True)
        ohm = iota == idxm
        d_ref[...] = jnp.where(ohm, jnp.inf, d)
        oh = jnp.where(ohm, 1.0, 0.0)
        gxyz = jnp.dot(oh, xyzt, preferred_element_type=jnp.float32)
        rel = gxyz - c_m
        if feat_is_xyz:
            gfeat = gxyz
        else:
            gfeat = jnp.dot(oh, ptst_ref[0], preferred_element_type=jnp.float32)
        h = (jnp.dot(rel, wat_x, preferred_element_type=jnp.float32)
             + jnp.dot(gfeat, wat_f, preferred_element_type=jnp.float32) + ba)
        h = jnp.maximum(h, 0.0)
        h2 = jnp.dot(h, wbt, preferred_element_type=jnp.float32) + bb
        acc = h2 if acc is None else jnp.maximum(acc, h2)
    out_ref[0] = acc


def _sa_pallas(new_xyz, xyz, ptst, wa, ba, wb, bb, k, feat_is_xyz):
    Bn, _, M = new_xyz.shape
    Nn = xyz.shape[2]
    C = ptst.shape[2]
    C_hid = wa.shape[0]
    C_out = wb.shape[0]
    nxt = jnp.transpose(new_xyz, (0, 2, 1))
    xyzt = jnp.transpose(xyz, (0, 2, 1))
    wat_x = jnp.transpose(wa[:, :3])
    wat_f = jnp.transpose(wa[:, 3:])
    ba2 = ba.reshape(1, C_hid)
    wbt = jnp.transpose(wb)
    bb2 = bb.reshape(1, C_out)
    return pl.pallas_call(
        functools.partial(_sa_kernel, k=k, feat_is_xyz=feat_is_xyz),
        out_shape=jax.ShapeDtypeStruct((Bn, M, C_out), jnp.float32),
        grid=(Bn,),
        in_specs=[
            pl.BlockSpec((1, M, 3), lambda b: (b, 0, 0)),
            pl.BlockSpec((1, 3, Nn), lambda b: (b, 0, 0)),
            pl.BlockSpec((1, Nn, 3), lambda b: (b, 0, 0)),
            pl.BlockSpec((1, Nn, C), lambda b: (b, 0, 0)),
            pl.BlockSpec((3, C_hid), lambda b: (0, 0)),
            pl.BlockSpec((C, C_hid), lambda b: (0, 0)),
            pl.BlockSpec((1, C_hid), lambda b: (0, 0)),
            pl.BlockSpec((C_hid, C_out), lambda b: (0, 0)),
            pl.BlockSpec((1, C_out), lambda b: (0, 0)),
        ],
        out_specs=pl.BlockSpec((1, M, C_out), lambda b: (b, 0, 0)),
        scratch_shapes=[pltpu.VMEM((M, Nn), jnp.float32)],
    )(nxt, xyz, xyzt, ptst, wat_x, wat_f, ba2, wbt, bb2)


def kernel(point_cloud, W1, b1, W2, b2, W3, b3, W4, b4):
    nx1 = _fps_pallas(point_cloud, NP1)
    xyzt1 = jnp.transpose(point_cloud, (0, 2, 1))
    l1_pts_t = _sa_pallas(nx1, point_cloud, xyzt1, W1, b1, W2, b2, K1, True)
    nx2 = _fps_pallas(nx1, NP2)
    l2_pts_t = _sa_pallas(nx2, nx1, l1_pts_t, W3, b3, W4, b4, K2, False)
    return jnp.transpose(l2_pts_t, (0, 2, 1))
